# baseline (speedup 1.0000x reference)
import dataclasses
import functools

import jax
import jax.numpy as jnp
from jax import lax
from jax.experimental import pallas as pl
from jax.experimental.pallas import tpu as pltpu
from jax.experimental.pallas import tpu_sc as plsc

jax.config.update("jax_enable_x64", True)

_NUM_SEGMENTS = 42
_NUM_BINS = 5000
_NUM_INTERVAL = (_NUM_SEGMENTS + 1) * _NUM_BINS
_N = 2_000_000
_SHIFT = 0.9162907600402832
_STEP = 1.0 / _NUM_BINS

_BINS_PAD = 215_040
_NW_TAB = _BINS_PAD // 2
_WPT = _NW_TAB // 32

_NW = 32
_CH = 2048
_NCHUNK = 32
_PW = _CH * _NCHUNK
_N_PAD = _NW * _PW

_MESH = plsc.VectorSubcoreMesh(core_axis_name="c", subcore_axis_name="s")

_sc_params = pltpu.CompilerParams()
if "needs_layout_passes" in pltpu.CompilerParams.__dataclass_fields__:
    _sc_params = dataclasses.replace(_sc_params, needs_layout_passes=False)


def _bf16_bits_rne(x_f32):
    i = lax.bitcast_convert_type(x_f32, jnp.int32)
    odd = jnp.bitwise_and(lax.shift_right_logical(i, jnp.int32(16)), 1)
    return jnp.bitwise_and(lax.shift_right_logical(i + 0x7FFF + odd, jnp.int32(16)), 0xFFFF)


def _fold_body(pp_hbm, ep_hbm, tw_hbm, pbuf, ebuf, wbuf):
    f32, i32 = jnp.float32, jnp.int32
    wid = lax.axis_index("s") * i32(2) + lax.axis_index("c")
    iot = lax.iota(i32, 16)
    pltpu.sync_copy(pp_hbm.at[pl.ds(wid * i32(4 * _WPT), 4 * _WPT)], pbuf)
    pltpu.sync_copy(ep_hbm.at[pl.ds(wid * i32(4 * _WPT), 4 * _WPT)], ebuf)

    def _vec(k, _):
        b = (iot + k * i32(16)) * i32(4)
        i1 = b + i32(1)
        i3 = b + i32(3)
        pa = plsc.load_gather(pbuf, [i1])
        pb = plsc.load_gather(pbuf, [i3])
        ea = plsc.load_gather(ebuf, [i1])
        eb = plsc.load_gather(ebuf, [i3])
        aa = jnp.where(ea > f32(10000.0), f32(0.9995) * (pa / ea), f32(-1.0))
        ab = jnp.where(eb > f32(10000.0), f32(0.9995) * (pb / eb), f32(-1.0))
        w = jnp.bitwise_or(lax.shift_left(_bf16_bits_rne(ab), i32(16)),
                           _bf16_bits_rne(aa))
        wbuf[pl.ds(k * i32(16), 16)] = w
        return 0

    lax.fori_loop(i32(0), i32(_WPT // 16), _vec, 0)
    pltpu.sync_copy(wbuf, tw_hbm.at[pl.ds(wid * i32(_WPT), _WPT)])


_fold = functools.partial(
    pl.kernel,
    compiler_params=_sc_params,
    out_type=jax.ShapeDtypeStruct((_NW_TAB,), jnp.int32),
    mesh=_MESH,
    scratch_types=[
        pltpu.VMEM((4 * _WPT,), jnp.float32),
        pltpu.VMEM((4 * _WPT,), jnp.float32),
        pltpu.VMEM((_WPT,), jnp.int32),
    ],
)(_fold_body)


def _sc_body(lg_hbm, sv_hbm, ln_hbm, tw_hbm, out_hbm, tab, lbuf, svbuf, lnbuf, obuf):
    f32, i32 = jnp.float32, jnp.int32
    wid = lax.axis_index("s") * i32(2) + lax.axis_index("c")
    base = wid * i32(_PW)
    iot = lax.iota(i32, 16)
    pltpu.sync_copy(tw_hbm, tab)

    def _chunk(ci, _):
        off = base + ci * i32(_CH)
        pltpu.sync_copy(lg_hbm.at[pl.ds(off, _CH)], lbuf)
        pltpu.sync_copy(sv_hbm.at[pl.ds(off * i32(2), 2 * _CH)], svbuf)
        pltpu.sync_copy(ln_hbm.at[pl.ds(off * i32(2), 2 * _CH)], lnbuf)

        def _vec(k, _):
            j = k * i32(16)
            idx2 = (iot + j) * i32(2)
            x = lbuf[pl.ds(j, 16)] - f32(_SHIFT)
            p = f32(1.0) / (f32(1.0) + jnp.exp(-x))
            y = p / f32(_STEP)
            ti = y.astype(i32)
            ceil_y = ti + (y > ti.astype(f32)).astype(i32)
            bidx = ceil_y - 1
            sv = plsc.load_gather(svbuf, [idx2]) + 1
            ln = plsc.load_gather(lnbuf, [idx2])
            ok = (ln == 1) & (sv >= 0) & (sv <= _NUM_SEGMENTS)
            s = jnp.where(ok, sv, 0)
            ids = bidx + s * _NUM_BINS
            ids = jnp.minimum(jnp.maximum(ids, 0), _NUM_INTERVAL - 1)
            w = plsc.load_gather(tab, [lax.shift_right_logical(ids, i32(1))])
            half = jnp.where(jnp.bitwise_and(ids, 1) == 1,
                             lax.shift_right_logical(w, i32(16)), w)
            g = lax.bitcast_convert_type(lax.shift_left(half, i32(16)), f32)
            obuf[pl.ds(j, 16)] = jnp.where(g < f32(0.0), p, g + f32(0.0005) * p)
            return 0

        lax.fori_loop(i32(0), i32(_CH // 16), _vec, 0)
        pltpu.sync_copy(obuf, out_hbm.at[pl.ds(off, _CH)])
        return 0

    lax.fori_loop(i32(0), i32(_NCHUNK), _chunk, 0)


_sc_calib = functools.partial(
    pl.kernel,
    compiler_params=_sc_params,
    out_type=jax.ShapeDtypeStruct((_N_PAD,), jnp.float32),
    mesh=_MESH,
    scratch_types=[
        pltpu.VMEM((_NW_TAB,), jnp.int32),
        pltpu.VMEM((_CH,), jnp.float32),
        pltpu.VMEM((2 * _CH,), jnp.int32),
        pltpu.VMEM((2 * _CH,), jnp.int32),
        pltpu.VMEM((_CH,), jnp.float32),
    ],
)(_sc_body)


def _pairs_f32(table_f64):
    v = lax.bitcast_convert_type(table_f64, jnp.float32)
    return jnp.pad(v, ((0, _BINS_PAD - _NUM_INTERVAL), (0, 0))).reshape(-1)


def _pairs_i32(x_i64_flat):
    v = lax.bitcast_convert_type(x_i64_flat, jnp.int32)
    return jnp.pad(v, ((0, _N_PAD - _N), (0, 0))).reshape(-1)


def kernel(segment_value, segment_lengths, logit, bin_num_positives, bin_num_examples):
    tw = _fold(_pairs_f32(bin_num_positives), _pairs_f32(bin_num_examples))
    lg = jnp.pad(logit.reshape(-1), (0, _N_PAD - _N))
    sv = _pairs_i32(segment_value)
    ln = _pairs_i32(segment_lengths.reshape(-1))
    out = _sc_calib(lg, sv, ln, tw)
    return out[:_N].reshape(-1, 1)


# speedup vs baseline: 7.5825x; 7.5825x over previous
import dataclasses
import functools

import jax
import jax.numpy as jnp
from jax import lax
from jax.experimental import pallas as pl
from jax.experimental.pallas import tpu as pltpu
from jax.experimental.pallas import tpu_sc as plsc

jax.config.update("jax_enable_x64", True)

_NUM_SEGMENTS = 42
_NUM_BINS = 5000
_NUM_INTERVAL = (_NUM_SEGMENTS + 1) * _NUM_BINS
_N = 2_000_000
_SHIFT = 0.9162907600402832
_STEP = 1.0 / _NUM_BINS

_BINS_PAD = 215_040
_NW_TAB = _BINS_PAD // 2
_WPT = _NW_TAB // 32

_NW = 32
_CH = 2048
_NCHUNK = 32
_PW = _CH * _NCHUNK
_N_PAD = _NW * _PW

_MESH = plsc.VectorSubcoreMesh(core_axis_name="c", subcore_axis_name="s")

_sc_params = pltpu.CompilerParams()
if "needs_layout_passes" in pltpu.CompilerParams.__dataclass_fields__:
    _sc_params = dataclasses.replace(_sc_params, needs_layout_passes=False)


def _bf16_bits_rne(x_f32):
    i = lax.bitcast_convert_type(x_f32, jnp.int32)
    odd = jnp.bitwise_and(lax.shift_right_logical(i, jnp.int32(16)), 1)
    return jnp.bitwise_and(lax.shift_right_logical(i + 0x7FFF + odd, jnp.int32(16)), 0xFFFF)


def _fold_body(pp_hbm, ep_hbm, tw_hbm, pbuf, ebuf, wbuf):
    f32, i32 = jnp.float32, jnp.int32
    wid = lax.axis_index("s") * i32(2) + lax.axis_index("c")
    iot = lax.iota(i32, 16)
    pltpu.sync_copy(pp_hbm.at[pl.ds(wid * i32(4 * _WPT), 4 * _WPT)], pbuf)
    pltpu.sync_copy(ep_hbm.at[pl.ds(wid * i32(4 * _WPT), 4 * _WPT)], ebuf)

    def _vec(k, _):
        b = (iot + k * i32(16)) * i32(4)
        i1 = b + i32(1)
        i3 = b + i32(3)
        pa = plsc.load_gather(pbuf, [i1])
        pb = plsc.load_gather(pbuf, [i3])
        ea = plsc.load_gather(ebuf, [i1])
        eb = plsc.load_gather(ebuf, [i3])
        aa = jnp.where(ea > f32(10000.0), f32(0.9995) * (pa / ea), f32(-1.0))
        ab = jnp.where(eb > f32(10000.0), f32(0.9995) * (pb / eb), f32(-1.0))
        w = jnp.bitwise_or(lax.shift_left(_bf16_bits_rne(ab), i32(16)),
                           _bf16_bits_rne(aa))
        wbuf[pl.ds(k * i32(16), 16)] = w
        return 0

    lax.fori_loop(i32(0), i32(_WPT // 16), _vec, 0)
    pltpu.sync_copy(wbuf, tw_hbm.at[pl.ds(wid * i32(_WPT), _WPT)])


_fold = functools.partial(
    pl.kernel,
    compiler_params=_sc_params,
    out_type=jax.ShapeDtypeStruct((_NW_TAB,), jnp.int32),
    mesh=_MESH,
    scratch_types=[
        pltpu.VMEM((4 * _WPT,), jnp.float32),
        pltpu.VMEM((4 * _WPT,), jnp.float32),
        pltpu.VMEM((_WPT,), jnp.int32),
    ],
)(_fold_body)


def _sc_body(lg_hbm, sv_hbm, ln_hbm, tw_hbm, out_hbm, tab, lbuf, svbuf, lnbuf, obuf):
    f32, i32 = jnp.float32, jnp.int32
    wid = lax.axis_index("s") * i32(2) + lax.axis_index("c")
    base = wid * i32(_PW)
    iot = lax.iota(i32, 16)
    pltpu.sync_copy(tw_hbm, tab)

    def _chunk(ci, _):
        off = base + ci * i32(_CH)
        pltpu.sync_copy(lg_hbm.at[pl.ds(off, _CH)], lbuf)
        pltpu.sync_copy(sv_hbm.at[pl.ds(off, _CH)], svbuf)
        pltpu.sync_copy(ln_hbm.at[pl.ds(off, _CH)], lnbuf)

        def _vec(k, _):
            j = k * i32(16)
            x = lbuf[pl.ds(j, 16)] - f32(_SHIFT)
            p = f32(1.0) / (f32(1.0) + jnp.exp(-x))
            y = p / f32(_STEP)
            ti = y.astype(i32)
            ceil_y = ti + (y > ti.astype(f32)).astype(i32)
            bidx = ceil_y - 1
            sv = svbuf[pl.ds(j, 16)] + 1
            ln = lnbuf[pl.ds(j, 16)]
            ok = (ln == 1) & (sv >= 0) & (sv <= _NUM_SEGMENTS)
            s = jnp.where(ok, sv, 0)
            ids = bidx + s * _NUM_BINS
            ids = jnp.minimum(jnp.maximum(ids, 0), _NUM_INTERVAL - 1)
            w = plsc.load_gather(tab, [lax.shift_right_logical(ids, i32(1))])
            half = jnp.where(jnp.bitwise_and(ids, 1) == 1,
                             lax.shift_right_logical(w, i32(16)), w)
            g = lax.bitcast_convert_type(lax.shift_left(half, i32(16)), f32)
            obuf[pl.ds(j, 16)] = jnp.where(g < f32(0.0), p, g + f32(0.0005) * p)
            return 0

        lax.fori_loop(i32(0), i32(_CH // 16), _vec, 0)
        pltpu.sync_copy(obuf, out_hbm.at[pl.ds(off, _CH)])
        return 0

    lax.fori_loop(i32(0), i32(_NCHUNK), _chunk, 0)


_sc_calib = functools.partial(
    pl.kernel,
    compiler_params=_sc_params,
    out_type=jax.ShapeDtypeStruct((_N_PAD,), jnp.float32),
    mesh=_MESH,
    scratch_types=[
        pltpu.VMEM((_NW_TAB,), jnp.int32),
        pltpu.VMEM((_CH,), jnp.float32),
        pltpu.VMEM((_CH,), jnp.int32),
        pltpu.VMEM((_CH,), jnp.int32),
        pltpu.VMEM((_CH,), jnp.float32),
    ],
)(_sc_body)


def _pairs_f32(table_f64):
    v = lax.bitcast_convert_type(table_f64, jnp.float32)
    return jnp.pad(v, ((0, _BINS_PAD - _NUM_INTERVAL), (0, 0))).reshape(-1)


def kernel(segment_value, segment_lengths, logit, bin_num_positives, bin_num_examples):
    tw = _fold(_pairs_f32(bin_num_positives), _pairs_f32(bin_num_examples))
    padn = _N_PAD - _N
    lg = jnp.pad(logit.reshape(-1), (0, padn))
    sv = jnp.pad(segment_value.astype(jnp.int32), (0, padn))
    ln = jnp.pad(segment_lengths.reshape(-1).astype(jnp.int32), (0, padn))
    out = _sc_calib(lg, sv, ln, tw)
    return out[:_N].reshape(-1, 1)


# speedup vs baseline: 9.3245x; 1.2297x over previous
import dataclasses
import functools

import jax
import jax.numpy as jnp
from jax import lax
from jax.experimental import pallas as pl
from jax.experimental.pallas import tpu as pltpu
from jax.experimental.pallas import tpu_sc as plsc

jax.config.update("jax_enable_x64", True)

_NUM_SEGMENTS = 42
_NUM_BINS = 5000
_NUM_INTERVAL = (_NUM_SEGMENTS + 1) * _NUM_BINS
_N = 2_000_000
_SHIFT = 0.9162907600402832
_STEP = 1.0 / _NUM_BINS

_NW_TAB = 107_520
_WPT = _NW_TAB // 32
_PAIR_N = 2 * _NUM_INTERVAL
_LAST_PAIRS = _PAIR_N - 31 * 4 * _WPT

_NW = 32
_CH = 2048
_S_LO = 62_496
_S_HI = _S_LO + 16
_NFULL = 30

_MESH = plsc.VectorSubcoreMesh(core_axis_name="c", subcore_axis_name="s")

_sc_params = pltpu.CompilerParams()
if "needs_layout_passes" in pltpu.CompilerParams.__dataclass_fields__:
    _sc_params = dataclasses.replace(_sc_params, needs_layout_passes=False)


def _bf16_bits_rne(x_f32):
    i = lax.bitcast_convert_type(x_f32, jnp.int32)
    odd = jnp.bitwise_and(lax.shift_right_logical(i, jnp.int32(16)), 1)
    return jnp.bitwise_and(lax.shift_right_logical(i + 0x7FFF + odd, jnp.int32(16)), 0xFFFF)


def _fold_body(pp_hbm, ep_hbm, tw_hbm, pbuf, ebuf, wbuf):
    f32, i32 = jnp.float32, jnp.int32
    wid = lax.axis_index("s") * i32(2) + lax.axis_index("c")
    iot = lax.iota(i32, 16)
    src = wid * i32(4 * _WPT)

    @pl.when(wid < _NW - 1)
    def _():
        pltpu.sync_copy(pp_hbm.at[pl.ds(src, 4 * _WPT)], pbuf)
        pltpu.sync_copy(ep_hbm.at[pl.ds(src, 4 * _WPT)], ebuf)

    @pl.when(wid == _NW - 1)
    def _():
        pltpu.sync_copy(pp_hbm.at[pl.ds(src, _LAST_PAIRS)], pbuf.at[pl.ds(0, _LAST_PAIRS)])
        pltpu.sync_copy(ep_hbm.at[pl.ds(src, _LAST_PAIRS)], ebuf.at[pl.ds(0, _LAST_PAIRS)])

    def _vec(k, _):
        b = (iot + k * i32(16)) * i32(4)
        i1 = b + i32(1)
        i3 = b + i32(3)
        pa = plsc.load_gather(pbuf, [i1])
        pb = plsc.load_gather(pbuf, [i3])
        ea = plsc.load_gather(ebuf, [i1])
        eb = plsc.load_gather(ebuf, [i3])
        aa = jnp.where(ea > f32(10000.0), f32(0.9995) * (pa / ea), f32(-1.0))
        ab = jnp.where(eb > f32(10000.0), f32(0.9995) * (pb / eb), f32(-1.0))
        w = jnp.bitwise_or(lax.shift_left(_bf16_bits_rne(ab), i32(16)),
                           _bf16_bits_rne(aa))
        wbuf[pl.ds(k * i32(16), 16)] = w
        return 0

    lax.fori_loop(i32(0), i32(_WPT // 16), _vec, 0)
    pltpu.sync_copy(wbuf, tw_hbm.at[pl.ds(wid * i32(_WPT), _WPT)])


_fold = functools.partial(
    pl.kernel,
    compiler_params=_sc_params,
    out_type=jax.ShapeDtypeStruct((_NW_TAB,), jnp.int32),
    mesh=_MESH,
    scratch_types=[
        pltpu.VMEM((4 * _WPT,), jnp.float32),
        pltpu.VMEM((4 * _WPT,), jnp.float32),
        pltpu.VMEM((_WPT,), jnp.int32),
    ],
)(_fold_body)


def _sc_body(lg_hbm, sv_hbm, ln_hbm, tw_hbm, out_hbm, tab, lbuf, svbuf, lnbuf, obuf):
    f32, i32 = jnp.float32, jnp.int32
    wid = lax.axis_index("s") * i32(2) + lax.axis_index("c")
    base = wid * i32(_S_LO) + jnp.minimum(wid, i32(8)) * i32(16)
    span = jnp.where(wid < 8, i32(_S_HI), i32(_S_LO))
    tail_off = base + span - i32(_CH)
    iot = lax.iota(i32, 16)
    pltpu.sync_copy(tw_hbm, tab)

    def _chunk(ci, _):
        off = jnp.where(ci == _NFULL, tail_off, base + ci * i32(_CH))
        pltpu.sync_copy(lg_hbm.at[pl.ds(off, _CH)], lbuf)
        pltpu.sync_copy(sv_hbm.at[pl.ds(off, _CH)], svbuf)
        pltpu.sync_copy(ln_hbm.at[pl.ds(off, _CH)], lnbuf)

        def _vec(k, _):
            j = k * i32(16)
            x = lbuf[pl.ds(j, 16)] - f32(_SHIFT)
            p = f32(1.0) / (f32(1.0) + jnp.exp(-x))
            y = p / f32(_STEP)
            ti = y.astype(i32)
            ceil_y = ti + (y > ti.astype(f32)).astype(i32)
            bidx = ceil_y - 1
            sv = svbuf[pl.ds(j, 16)] + 1
            ln = lnbuf[pl.ds(j, 16)]
            ok = (ln == 1) & (sv >= 0) & (sv <= _NUM_SEGMENTS)
            s = jnp.where(ok, sv, 0)
            ids = bidx + s * _NUM_BINS
            ids = jnp.minimum(jnp.maximum(ids, 0), _NUM_INTERVAL - 1)
            w = plsc.load_gather(tab, [lax.shift_right_logical(ids, i32(1))])
            half = jnp.where(jnp.bitwise_and(ids, 1) == 1,
                             lax.shift_right_logical(w, i32(16)), w)
            g = lax.bitcast_convert_type(lax.shift_left(half, i32(16)), f32)
            obuf[pl.ds(j, 16)] = jnp.where(g < f32(0.0), p, g + f32(0.0005) * p)
            return 0

        lax.fori_loop(i32(0), i32(_CH // 16), _vec, 0)
        pltpu.sync_copy(obuf, out_hbm.at[pl.ds(off, _CH)])
        return 0

    lax.fori_loop(i32(0), i32(_NFULL + 1), _chunk, 0)


_sc_calib = functools.partial(
    pl.kernel,
    compiler_params=_sc_params,
    out_type=jax.ShapeDtypeStruct((_N,), jnp.float32),
    mesh=_MESH,
    scratch_types=[
        pltpu.VMEM((_NW_TAB,), jnp.int32),
        pltpu.VMEM((_CH,), jnp.float32),
        pltpu.VMEM((_CH,), jnp.int32),
        pltpu.VMEM((_CH,), jnp.int32),
        pltpu.VMEM((_CH,), jnp.float32),
    ],
)(_sc_body)


def _pairs_f32(table_f64):
    return lax.bitcast_convert_type(table_f64, jnp.float32).reshape(-1)


def kernel(segment_value, segment_lengths, logit, bin_num_positives, bin_num_examples):
    tw = _fold(_pairs_f32(bin_num_positives), _pairs_f32(bin_num_examples))
    lg = logit.reshape(-1)
    sv = segment_value.astype(jnp.int32)
    ln = segment_lengths.reshape(-1).astype(jnp.int32)
    out = _sc_calib(lg, sv, ln, tw)
    return out.reshape(-1, 1)


# speedup vs baseline: 14.2776x; 1.5312x over previous
import dataclasses
import functools

import jax
import jax.numpy as jnp
from jax import lax
from jax.experimental import pallas as pl
from jax.experimental.pallas import tpu as pltpu
from jax.experimental.pallas import tpu_sc as plsc

jax.config.update("jax_enable_x64", True)

_NUM_SEGMENTS = 42
_NUM_BINS = 5000
_NUM_INTERVAL = (_NUM_SEGMENTS + 1) * _NUM_BINS
_N = 2_000_000
_SHIFT = 0.9162907600402832
_STEP = 1.0 / _NUM_BINS

_NW_TAB = 107_520
_WPT = _NW_TAB // 32
_BPT = 2 * _WPT
_LAST_BINS = _NUM_INTERVAL - 31 * _BPT

_NW = 32
_CH = 2048
_S_LO = 62_496
_S_HI = _S_LO + 16
_NFULL = 30

_MESH = plsc.VectorSubcoreMesh(core_axis_name="c", subcore_axis_name="s")

_sc_params = pltpu.CompilerParams()
for _fld, _val in (("needs_layout_passes", False), ("use_tc_tiling_on_sc", False)):
    if _fld in pltpu.CompilerParams.__dataclass_fields__:
        _sc_params = dataclasses.replace(_sc_params, **{_fld: _val})


def _bf16_bits_rne(x_f32):
    i = lax.bitcast_convert_type(x_f32, jnp.int32)
    odd = jnp.bitwise_and(lax.shift_right_logical(i, jnp.int32(16)), 1)
    return jnp.bitwise_and(lax.shift_right_logical(i + 0x7FFF + odd, jnp.int32(16)), 0xFFFF)


def _fold_body(pos_hbm, ex_hbm, tw_hbm, pbuf, ebuf, wbuf):
    f32, i32 = jnp.float32, jnp.int32
    wid = lax.axis_index("s") * i32(2) + lax.axis_index("c")
    iot = lax.iota(i32, 16)
    src = wid * i32(_BPT)

    @pl.when(wid < _NW - 1)
    def _():
        pltpu.sync_copy(pos_hbm.at[pl.ds(src, _BPT)], pbuf)
        pltpu.sync_copy(ex_hbm.at[pl.ds(src, _BPT)], ebuf)

    @pl.when(wid == _NW - 1)
    def _():
        pltpu.sync_copy(pos_hbm.at[pl.ds(src, _LAST_BINS)], pbuf.at[pl.ds(0, _LAST_BINS)])
        pltpu.sync_copy(ex_hbm.at[pl.ds(src, _LAST_BINS)], ebuf.at[pl.ds(0, _LAST_BINS)])

    def _vec(k, _):
        b = iot * i32(2) + k * i32(32)
        b1 = b + i32(1)
        pa = plsc.load_gather(pbuf, [b])
        pb = plsc.load_gather(pbuf, [b1])
        ea = plsc.load_gather(ebuf, [b])
        eb = plsc.load_gather(ebuf, [b1])
        aa = jnp.where(ea > f32(10000.0), f32(0.9995) * (pa / ea), f32(-1.0))
        ab = jnp.where(eb > f32(10000.0), f32(0.9995) * (pb / eb), f32(-1.0))
        w = jnp.bitwise_or(lax.shift_left(_bf16_bits_rne(ab), i32(16)),
                           _bf16_bits_rne(aa))
        wbuf[pl.ds(k * i32(16), 16)] = w
        return 0

    lax.fori_loop(i32(0), i32(_WPT // 16), _vec, 0)
    pltpu.sync_copy(wbuf, tw_hbm.at[pl.ds(wid * i32(_WPT), _WPT)])


_fold = functools.partial(
    pl.kernel,
    compiler_params=_sc_params,
    out_type=jax.ShapeDtypeStruct((_NW_TAB,), jnp.int32),
    mesh=_MESH,
    scratch_types=[
        pltpu.VMEM((_BPT,), jnp.float32),
        pltpu.VMEM((_BPT,), jnp.float32),
        pltpu.VMEM((_WPT,), jnp.int32),
    ],
)(_fold_body)


def _sc_body(lg_hbm, sv_hbm, ln_hbm, tw_hbm, out_hbm, tab, lbuf, svbuf, lnbuf, obuf):
    f32, i32 = jnp.float32, jnp.int32
    wid = lax.axis_index("s") * i32(2) + lax.axis_index("c")
    base = wid * i32(_S_LO) + jnp.minimum(wid, i32(8)) * i32(16)
    span = jnp.where(wid < 8, i32(_S_HI), i32(_S_LO))
    tail_off = base + span - i32(_CH)
    iot = lax.iota(i32, 16)
    zeros = iot * i32(0)
    pltpu.sync_copy(tw_hbm, tab)

    def _chunk(ci, _):
        off = jnp.where(ci == _NFULL, tail_off, base + ci * i32(_CH))
        pltpu.sync_copy(lg_hbm.at[pl.ds(off, _CH)], lbuf)
        pltpu.sync_copy(sv_hbm.at[pl.ds(off, _CH)], svbuf)
        pltpu.sync_copy(ln_hbm.at[pl.ds(off, _CH)], lnbuf)

        def _vec(k, _):
            j = k * i32(16)
            rows = iot + j
            x = lbuf[pl.ds(j, 16)] - f32(_SHIFT)
            p = f32(1.0) / (f32(1.0) + jnp.exp(-x))
            y = p / f32(_STEP)
            ti = y.astype(i32)
            ceil_y = ti + (y > ti.astype(f32)).astype(i32)
            bidx = ceil_y - 1
            sv = svbuf[pl.ds(j, 16)] + 1
            ln = lnbuf[pl.ds(j, 16)]
            ok = (ln == 1) & (sv >= 0) & (sv <= _NUM_SEGMENTS)
            s = jnp.where(ok, sv, 0)
            ids = bidx + s * _NUM_BINS
            ids = jnp.minimum(jnp.maximum(ids, 0), _NUM_INTERVAL - 1)
            w = plsc.load_gather(tab, [lax.shift_right_logical(ids, i32(1))])
            half = jnp.where(jnp.bitwise_and(ids, 1) == 1,
                             lax.shift_right_logical(w, i32(16)), w)
            g = lax.bitcast_convert_type(lax.shift_left(half, i32(16)), f32)
            res = jnp.where(g < f32(0.0), p, g + f32(0.0005) * p)
            obuf[pl.ds(j, 16)] = res
            return 0

        lax.fori_loop(i32(0), i32(_CH // 16), _vec, 0)
        pltpu.sync_copy(obuf, out_hbm.at[pl.ds(off, _CH)])
        return 0

    lax.fori_loop(i32(0), i32(_NFULL + 1), _chunk, 0)


_sc_calib = functools.partial(
    pl.kernel,
    compiler_params=_sc_params,
    out_type=jax.ShapeDtypeStruct((_N,), jnp.float32),
    mesh=_MESH,
    scratch_types=[
        pltpu.VMEM((_NW_TAB,), jnp.int32),
        pltpu.VMEM((_CH,), jnp.float32),
        pltpu.VMEM((_CH,), jnp.int32),
        pltpu.VMEM((_CH,), jnp.int32),
        pltpu.VMEM((_CH,), jnp.float32),
    ],
)(_sc_body)


def kernel(segment_value, segment_lengths, logit, bin_num_positives, bin_num_examples):
    pos32 = bin_num_positives.astype(jnp.float32)
    ex32 = bin_num_examples.astype(jnp.float32)
    tw = _fold(pos32, ex32)
    sv = lax.bitcast_convert_type(segment_value.astype(jnp.uint32), jnp.int32)
    ln = lax.bitcast_convert_type(segment_lengths.reshape(-1).astype(jnp.uint32), jnp.int32)
    return _sc_calib(logit.reshape(-1), sv, ln, tw).reshape(-1, 1)


# speedup vs baseline: 16.3127x; 1.1425x over previous
import dataclasses
import functools

import jax
import jax.numpy as jnp
from jax import lax
from jax.experimental import pallas as pl
from jax.experimental.pallas import tpu as pltpu
from jax.experimental.pallas import tpu_sc as plsc

jax.config.update("jax_enable_x64", True)

_NUM_SEGMENTS = 42
_NUM_BINS = 5000
_NUM_INTERVAL = (_NUM_SEGMENTS + 1) * _NUM_BINS
_N = 2_000_000
_SHIFT = 0.9162907600402832
_STEP = 1.0 / _NUM_BINS

_NW_TAB = 107_520
_WPT = _NW_TAB // 32
_BPT = 2 * _WPT
_LAST_BINS = _NUM_INTERVAL - 31 * _BPT

_NW = 32
_CH = 2048
_S_LO = 62_496
_S_HI = _S_LO + 16
_NFULL = 30

_MESH = plsc.VectorSubcoreMesh(core_axis_name="c", subcore_axis_name="s")

_sc_params = pltpu.CompilerParams()
for _fld, _val in (("needs_layout_passes", False), ("use_tc_tiling_on_sc", False)):
    if _fld in pltpu.CompilerParams.__dataclass_fields__:
        _sc_params = dataclasses.replace(_sc_params, **{_fld: _val})


def _bf16_bits_rne(x_f32):
    i = lax.bitcast_convert_type(x_f32, jnp.int32)
    odd = jnp.bitwise_and(lax.shift_right_logical(i, jnp.int32(16)), 1)
    return jnp.bitwise_and(lax.shift_right_logical(i + 0x7FFF + odd, jnp.int32(16)), 0xFFFF)


def _fold_body(pos_hbm, ex_hbm, tw_hbm, pbuf, ebuf, wbuf):
    f32, i32 = jnp.float32, jnp.int32
    wid = lax.axis_index("s") * i32(2) + lax.axis_index("c")
    iot = lax.iota(i32, 16)
    src = wid * i32(_BPT)

    @pl.when(wid < _NW - 1)
    def _():
        pltpu.sync_copy(pos_hbm.at[pl.ds(src, _BPT)], pbuf)
        pltpu.sync_copy(ex_hbm.at[pl.ds(src, _BPT)], ebuf)

    @pl.when(wid == _NW - 1)
    def _():
        pltpu.sync_copy(pos_hbm.at[pl.ds(src, _LAST_BINS)], pbuf.at[pl.ds(0, _LAST_BINS)])
        pltpu.sync_copy(ex_hbm.at[pl.ds(src, _LAST_BINS)], ebuf.at[pl.ds(0, _LAST_BINS)])

    def _vec(k, _):
        b = iot * i32(2) + k * i32(32)
        b1 = b + i32(1)
        pa = plsc.load_gather(pbuf, [b])
        pb = plsc.load_gather(pbuf, [b1])
        ea = plsc.load_gather(ebuf, [b])
        eb = plsc.load_gather(ebuf, [b1])
        aa = jnp.where(ea > f32(10000.0), f32(0.9995) * (pa / ea), f32(-1.0))
        ab = jnp.where(eb > f32(10000.0), f32(0.9995) * (pb / eb), f32(-1.0))
        w = jnp.bitwise_or(lax.shift_left(_bf16_bits_rne(ab), i32(16)),
                           _bf16_bits_rne(aa))
        wbuf[pl.ds(k * i32(16), 16)] = w
        return 0

    lax.fori_loop(i32(0), i32(_WPT // 16), _vec, 0)
    pltpu.sync_copy(wbuf, tw_hbm.at[pl.ds(wid * i32(_WPT), _WPT)])


_fold = functools.partial(
    pl.kernel,
    compiler_params=_sc_params,
    out_type=jax.ShapeDtypeStruct((_NW_TAB,), jnp.int32),
    mesh=_MESH,
    scratch_types=[
        pltpu.VMEM((_BPT,), jnp.float32),
        pltpu.VMEM((_BPT,), jnp.float32),
        pltpu.VMEM((_WPT,), jnp.int32),
    ],
)(_fold_body)


def _sc_body(lg_hbm, sv_hbm, ln_hbm, tw_hbm, out_hbm, tab,
             lbuf0, svbuf0, lnbuf0, lbuf1, svbuf1, lnbuf1, obuf, sem0, sem1):
    f32, i32 = jnp.float32, jnp.int32
    wid = lax.axis_index("s") * i32(2) + lax.axis_index("c")
    base = wid * i32(_S_LO) + jnp.minimum(wid, i32(8)) * i32(16)
    span = jnp.where(wid < 8, i32(_S_HI), i32(_S_LO))
    tail_off = base + span - i32(_CH)
    set0 = (lbuf0, svbuf0, lnbuf0, sem0)
    set1 = (lbuf1, svbuf1, lnbuf1, sem1)

    def _off(ci):
        return jnp.where(ci == i32(_NFULL), tail_off, base + ci * i32(_CH))

    def _start(ci, bufs):
        off = _off(ci)
        pltpu.async_copy(lg_hbm.at[pl.ds(off, _CH)], bufs[0], bufs[3])
        pltpu.async_copy(sv_hbm.at[pl.ds(off, _CH)], bufs[1], bufs[3])
        pltpu.async_copy(ln_hbm.at[pl.ds(off, _CH)], bufs[2], bufs[3])

    def _wait(ci, bufs):
        off = _off(ci)
        pltpu.make_async_copy(lg_hbm.at[pl.ds(off, _CH)], bufs[0], bufs[3]).wait()
        pltpu.make_async_copy(sv_hbm.at[pl.ds(off, _CH)], bufs[1], bufs[3]).wait()
        pltpu.make_async_copy(ln_hbm.at[pl.ds(off, _CH)], bufs[2], bufs[3]).wait()

    def _compute(ci, bufs):
        lbuf, svbuf, lnbuf = bufs[0], bufs[1], bufs[2]

        def _vec(k, _):
            j = k * i32(16)
            x = lbuf[pl.ds(j, 16)] - f32(_SHIFT)
            p = f32(1.0) / (f32(1.0) + jnp.exp(-x))
            y = p / f32(_STEP)
            ti = y.astype(i32)
            ceil_y = ti + (y > ti.astype(f32)).astype(i32)
            bidx = ceil_y - 1
            sv = svbuf[pl.ds(j, 16)] + 1
            ln = lnbuf[pl.ds(j, 16)]
            ok = (ln == 1) & (sv >= 0) & (sv <= _NUM_SEGMENTS)
            s = jnp.where(ok, sv, 0)
            ids = bidx + s * _NUM_BINS
            ids = jnp.minimum(jnp.maximum(ids, 0), _NUM_INTERVAL - 1)
            w = plsc.load_gather(tab, [lax.shift_right_logical(ids, i32(1))])
            half = jnp.where(jnp.bitwise_and(ids, 1) == 1,
                             lax.shift_right_logical(w, i32(16)), w)
            g = lax.bitcast_convert_type(lax.shift_left(half, i32(16)), f32)
            res = jnp.where(g < f32(0.0), p, g + f32(0.0005) * p)
            obuf[pl.ds(j, 16)] = res
            return 0

        lax.fori_loop(i32(0), i32(_CH // 16), _vec, 0)
        pltpu.sync_copy(obuf, out_hbm.at[pl.ds(_off(ci), _CH)])

    _start(i32(0), set0)
    pltpu.sync_copy(tw_hbm, tab)

    def _pair(k, _):
        c0 = k * i32(2)
        c1 = c0 + i32(1)

        @pl.when(c1 <= i32(_NFULL))
        def _():
            _start(c1, set1)

        _wait(c0, set0)
        _compute(c0, set0)

        @pl.when(c0 + i32(2) <= i32(_NFULL))
        def _():
            _start(c0 + i32(2), set0)

        @pl.when(c1 <= i32(_NFULL))
        def _():
            _wait(c1, set1)
            _compute(c1, set1)

        return 0

    lax.fori_loop(i32(0), i32((_NFULL + 2) // 2), _pair, 0)


_sc_calib = functools.partial(
    pl.kernel,
    compiler_params=_sc_params,
    out_type=jax.ShapeDtypeStruct((_N,), jnp.float32),
    mesh=_MESH,
    scratch_types=[
        pltpu.VMEM((_NW_TAB,), jnp.int32),
        pltpu.VMEM((_CH,), jnp.float32),
        pltpu.VMEM((_CH,), jnp.int32),
        pltpu.VMEM((_CH,), jnp.int32),
        pltpu.VMEM((_CH,), jnp.float32),
        pltpu.VMEM((_CH,), jnp.int32),
        pltpu.VMEM((_CH,), jnp.int32),
        pltpu.VMEM((_CH,), jnp.float32),
        pltpu.SemaphoreType.DMA,
        pltpu.SemaphoreType.DMA,
    ],
)(_sc_body)


def kernel(segment_value, segment_lengths, logit, bin_num_positives, bin_num_examples):
    pos32 = bin_num_positives.astype(jnp.float32)
    ex32 = bin_num_examples.astype(jnp.float32)
    tw = _fold(pos32, ex32)
    sv = lax.bitcast_convert_type(segment_value.astype(jnp.uint32), jnp.int32)
    ln = lax.bitcast_convert_type(segment_lengths.reshape(-1).astype(jnp.uint32), jnp.int32)
    return _sc_calib(logit.reshape(-1), sv, ln, tw).reshape(-1, 1)
